# baseline (speedup 1.0000x reference)
import functools

import jax
import jax.numpy as jnp
from jax import lax
from jax.experimental import pallas as pl
from jax.experimental.pallas import tpu as pltpu
from jax.experimental.pallas import tpu_sc as plsc

EMB = 8
LSZ = 4
NW = 32
NJ = 512
REP = 4


def _vq_body(x_ref, wm2_ref, wt_ref, wn_ref, q_ref, p_ref):
    wm2 = wm2_ref[...]
    wt = wt_ref[...]
    wn = wn_ref[...]
    for l in range(LSZ):
        x = x_ref[l]
        dots = jax.lax.dot(wm2, x, preferred_element_type=jnp.float32)
        dist = dots + wn
        g = dist
        for k in (1, 2, 4):
            g = jnp.minimum(g, jnp.roll(g, k, axis=0))
        onehot = (dist == g).astype(jnp.float32)
        q = jax.lax.dot(wt, onehot, preferred_element_type=jnp.float32)
        q_ref[l] = q
        p_ref[l] = q


def _cb_sc_body(cbsrc_hbm, cb_hbm, buf, sem):
    wid = lax.axis_index("s") * 2 + lax.axis_index("c")
    for e in range(EMB):
        for r in range(REP):
            pltpu.sync_copy(cbsrc_hbm.at[pl.ds(e * 1024, 1024)],
                            buf.at[e, r])
    j0 = wid * (NJ // NW)
    copies = []
    for e in range(EMB):
        for blk in range((NJ // NW) // REP):
            row0 = e * NJ + j0 + blk * REP
            copies.append(pltpu.async_copy(
                buf.at[e], cb_hbm.at[pl.ds(row0, REP)], sem))
    for c in copies:
        c.wait()


def kernel(latent, W):
    B = latent.shape[0]
    xt = latent.transpose(1, 2, 0)
    wm2 = (-2.0) * W
    wt = W.T
    wn = jnp.sum(W * W, axis=1, keepdims=True)
    cbsrc = jnp.broadcast_to(W[:, :, None], (EMB, EMB, 128)).reshape(-1)

    C = 16384
    grid = (B // C,)
    qt, pt = pl.pallas_call(
        _vq_body,
        grid=grid,
        in_specs=[
            pl.BlockSpec((LSZ, EMB, C), lambda i: (0, 0, i)),
            pl.BlockSpec((EMB, EMB), lambda i: (0, 0)),
            pl.BlockSpec((EMB, EMB), lambda i: (0, 0)),
            pl.BlockSpec((EMB, 1), lambda i: (0, 0)),
        ],
        out_specs=[
            pl.BlockSpec((LSZ, EMB, C), lambda i: (0, 0, i)),
            pl.BlockSpec((LSZ, EMB, C), lambda i: (0, 0, i)),
        ],
        out_shape=[
            jax.ShapeDtypeStruct((LSZ, EMB, B), jnp.float32),
            jax.ShapeDtypeStruct((LSZ, EMB, B), jnp.float32),
        ],
    )(xt, wm2, wt, wn)

    cb_kernel = functools.partial(
        pl.kernel,
        out_type=jax.ShapeDtypeStruct((EMB * NJ, EMB * 128), jnp.float32),
        mesh=plsc.VectorSubcoreMesh(core_axis_name="c", subcore_axis_name="s"),
        scratch_types=[
            pltpu.VMEM((EMB, REP, EMB * 128), jnp.float32),
            pltpu.SemaphoreType.DMA,
        ],
    )(_cb_sc_body)
    cb4 = cb_kernel(cbsrc).reshape(EMB, NJ, EMB, 128)

    q = qt.transpose(2, 0, 1)
    p = pt.transpose(2, 0, 1)
    cb = cb4.transpose(0, 2, 1, 3).reshape(EMB, EMB, B).transpose(2, 0, 1)
    return (p, q, cb)


# speedup vs baseline: 1.4927x; 1.4927x over previous
import functools

import jax
import jax.numpy as jnp
from jax import lax
from jax.experimental import pallas as pl
from jax.experimental.pallas import tpu as pltpu
from jax.experimental.pallas import tpu_sc as plsc

EMB = 8
LSZ = 4
NW = 32
NJ = 512
REP = 8


def _vq_body(x_ref, wm2_ref, wt_ref, wn_ref, q_ref, p_ref):
    wm2 = wm2_ref[...]
    wt = wt_ref[...]
    wn = wn_ref[...]
    for l in range(LSZ):
        x = x_ref[l]
        dots = jax.lax.dot(wm2, x, preferred_element_type=jnp.float32)
        dist = dots + wn
        g = dist
        for k in (1, 2, 4):
            g = jnp.minimum(g, jnp.roll(g, k, axis=0))
        onehot = (dist == g).astype(jnp.float32)
        q = jax.lax.dot(wt, onehot, preferred_element_type=jnp.float32)
        q_ref[l] = q
        p_ref[l] = q


def _cb_sc_body(cbsrc_hbm, cb_hbm, buf, sem):
    wid = lax.axis_index("s") * 2 + lax.axis_index("c")
    pltpu.sync_copy(cbsrc_hbm, buf)
    j0 = wid * (NJ // NW)
    copies = []
    for e in range(EMB):
        for blk in range((NJ // NW) // REP):
            row0 = e * NJ + j0 + blk * REP
            copies.append(pltpu.async_copy(
                buf.at[e], cb_hbm.at[pl.ds(row0, REP)], sem))
    for c in copies:
        c.wait()


def kernel(latent, W):
    B = latent.shape[0]
    xt = latent.transpose(1, 2, 0)
    wm2 = (-2.0) * W
    wt = W.T
    wn = jnp.sum(W * W, axis=1, keepdims=True)
    cbsrc = jnp.broadcast_to(W[:, None, :, None],
                             (EMB, REP, EMB, 128)).reshape(EMB, REP, EMB * 128)

    C = 16384
    grid = (B // C,)
    qt, pt = pl.pallas_call(
        _vq_body,
        grid=grid,
        in_specs=[
            pl.BlockSpec((LSZ, EMB, C), lambda i: (0, 0, i)),
            pl.BlockSpec((EMB, EMB), lambda i: (0, 0)),
            pl.BlockSpec((EMB, EMB), lambda i: (0, 0)),
            pl.BlockSpec((EMB, 1), lambda i: (0, 0)),
        ],
        out_specs=[
            pl.BlockSpec((LSZ, EMB, C), lambda i: (0, 0, i)),
            pl.BlockSpec((LSZ, EMB, C), lambda i: (0, 0, i)),
        ],
        out_shape=[
            jax.ShapeDtypeStruct((LSZ, EMB, B), jnp.float32),
            jax.ShapeDtypeStruct((LSZ, EMB, B), jnp.float32),
        ],
    )(xt, wm2, wt, wn)

    cb_kernel = functools.partial(
        pl.kernel,
        out_type=jax.ShapeDtypeStruct((EMB * NJ, EMB * 128), jnp.float32),
        mesh=plsc.VectorSubcoreMesh(core_axis_name="c", subcore_axis_name="s"),
        scratch_types=[
            pltpu.VMEM((EMB, REP, EMB * 128), jnp.float32),
            pltpu.SemaphoreType.DMA,
        ],
    )(_cb_sc_body)
    cb4 = cb_kernel(cbsrc).reshape(EMB, NJ, EMB, 128)

    q = qt.transpose(2, 0, 1)
    p = pt.transpose(2, 0, 1)
    cb = cb4.transpose(0, 2, 1, 3).reshape(EMB, EMB, B).transpose(2, 0, 1)
    return (p, q, cb)


# speedup vs baseline: 4.9833x; 3.3385x over previous
import jax
import jax.numpy as jnp
from jax.experimental import pallas as pl
from jax.experimental.pallas import tpu as pltpu

EMB = 8
LSZ = 4
C = 16384


def _in_copy(x_hbm, xbuf, sems, i, slot):
    return pltpu.make_async_copy(
        x_hbm.at[:, :, pl.ds(i * C, C)], xbuf.at[slot], sems.at[slot])


def _vq_body(x_hbm, wm2_ref, wt_ref, wn_ref, q_ref, p_ref, cb_ref,
             xbuf, sems):
    i = pl.program_id(0)
    n = pl.num_programs(0)
    slot = jax.lax.rem(i, 2)
    nxt = 1 - slot

    @pl.when(i == 0)
    def _():
        _in_copy(x_hbm, xbuf, sems, i, slot).start()

    @pl.when(i + 1 < n)
    def _():
        _in_copy(x_hbm, xbuf, sems, i + 1, nxt).start()

    _in_copy(x_hbm, xbuf, sems, i, slot).wait()

    wm2 = wm2_ref[...]
    wt = wt_ref[...]
    wn = wn_ref[...]
    for l in range(LSZ):
        x = xbuf[slot, l]
        dots = jax.lax.dot(wm2, x, preferred_element_type=jnp.float32)
        dist = dots + wn
        g = dist
        for k in (1, 2, 4):
            g = jnp.minimum(g, jnp.roll(g, k, axis=0))
        onehot = (dist == g).astype(jnp.float32)
        q = jax.lax.dot(wt, onehot, preferred_element_type=jnp.float32)
        q_ref[l] = q
        p_ref[l] = q
    for e in range(EMB):
        cb_ref[e] = jnp.broadcast_to(wt[:, e:e + 1], cb_ref.shape[1:])


def kernel(latent, W):
    B = latent.shape[0]
    xt = latent.transpose(1, 2, 0)
    wm2 = (-2.0) * W
    wt = W.T
    wn = jnp.sum(W * W, axis=1, keepdims=True)

    grid = (B // C,)
    qt, pt, cbt = pl.pallas_call(
        _vq_body,
        grid=grid,
        in_specs=[
            pl.BlockSpec(memory_space=pl.ANY),
            pl.BlockSpec((EMB, EMB), lambda i: (0, 0)),
            pl.BlockSpec((EMB, EMB), lambda i: (0, 0)),
            pl.BlockSpec((EMB, 1), lambda i: (0, 0)),
        ],
        out_specs=[
            pl.BlockSpec((LSZ, EMB, C), lambda i: (0, 0, i)),
            pl.BlockSpec((LSZ, EMB, C), lambda i: (0, 0, i)),
            pl.BlockSpec((EMB, EMB, C), lambda i: (0, 0, i)),
        ],
        out_shape=[
            jax.ShapeDtypeStruct((LSZ, EMB, B), jnp.float32),
            jax.ShapeDtypeStruct((LSZ, EMB, B), jnp.float32),
            jax.ShapeDtypeStruct((EMB, EMB, B), jnp.float32),
        ],
        scratch_shapes=[
            pltpu.VMEM((2, LSZ, EMB, C), jnp.float32),
            pltpu.SemaphoreType.DMA((2,)),
        ],
    )(xt, wm2, wt, wn)

    q = qt.transpose(2, 0, 1)
    p = pt.transpose(2, 0, 1)
    cb = cbt.transpose(2, 0, 1)
    return (p, q, cb)


# speedup vs baseline: 5.2050x; 1.0445x over previous
import jax
import jax.numpy as jnp
from jax.experimental import pallas as pl

EMB = 8
LSZ = 4


def _vq_body(x_ref, wm2_ref, wt_ref, wn_ref, q_ref, p_ref, cb_ref):
    wm2 = wm2_ref[...]
    wt = wt_ref[...]
    wn = wn_ref[...]
    for l in range(LSZ):
        x = x_ref[l]
        dots = jax.lax.dot(wm2, x, preferred_element_type=jnp.float32)
        dist = dots + wn
        g = dist
        for k in (1, 2, 4):
            g = jnp.minimum(g, jnp.roll(g, k, axis=0))
        onehot = (dist == g).astype(jnp.float32)
        q = jax.lax.dot(wt, onehot, preferred_element_type=jnp.float32)
        q_ref[l] = q
        p_ref[l] = q
    for e in range(EMB):
        cb_ref[e] = jnp.broadcast_to(wt[:, e:e + 1], cb_ref.shape[1:])


def kernel(latent, W):
    B = latent.shape[0]
    xt = latent.transpose(1, 2, 0)
    wm2 = (-2.0) * W
    wt = W.T
    wn = jnp.sum(W * W, axis=1, keepdims=True)

    C = 16384
    grid = (B // C,)
    qt, pt, cbt = pl.pallas_call(
        _vq_body,
        grid=grid,
        in_specs=[
            pl.BlockSpec((LSZ, EMB, C), lambda i: (0, 0, i)),
            pl.BlockSpec((EMB, EMB), lambda i: (0, 0)),
            pl.BlockSpec((EMB, EMB), lambda i: (0, 0)),
            pl.BlockSpec((EMB, 1), lambda i: (0, 0)),
        ],
        out_specs=[
            pl.BlockSpec((LSZ, EMB, C), lambda i: (0, 0, i)),
            pl.BlockSpec((LSZ, EMB, C), lambda i: (0, 0, i)),
            pl.BlockSpec((EMB, EMB, C), lambda i: (0, 0, i)),
        ],
        out_shape=[
            jax.ShapeDtypeStruct((LSZ, EMB, B), jnp.float32),
            jax.ShapeDtypeStruct((LSZ, EMB, B), jnp.float32),
            jax.ShapeDtypeStruct((EMB, EMB, B), jnp.float32),
        ],
    )(xt, wm2, wt, wn)

    q = qt.transpose(2, 0, 1)
    p = pt.transpose(2, 0, 1)
    cb = cbt.transpose(2, 0, 1)
    return (p, q, cb)
